# Initial kernel scaffold; baseline (speedup 1.0000x reference)
import jax
import jax.numpy as jnp
from jax.experimental import pallas as pl


def kernel(features, edge_index, W1, b1, W2, b2, Wo, bo):
    raise NotImplementedError("write your pallas kernel here")


# speedup vs baseline: 11.3764x; 11.3764x over previous
import functools

import jax
import jax.numpy as jnp
from jax import lax
from jax.experimental import pallas as pl
from jax.experimental.pallas import tpu as pltpu
from jax.experimental.pallas import tpu_sc as plsc

_N = 100000
_E = 6400000
_L = 128
_R = _E // _L
_NC = 2
_NS = 16
_NW = _NC * _NS
_RPW = -(-_R // _NW)
_BLK = 1024
_NP = ((_N + _BLK - 1) // _BLK) * _BLK
_GRID = _NP // _BLK
_NPT = _NP // _NS


def _sc_mesh():
    return plsc.VectorSubcoreMesh(core_axis_name="c", subcore_axis_name="s")


@functools.cache
def _get_sc_degrees():
    return functools.partial(
        pl.kernel,
        out_type=[jax.ShapeDtypeStruct((_NC, _NP), jnp.float32),
                  jax.ShapeDtypeStruct((_NC, _NP), jnp.float32)],
        mesh=_sc_mesh(),
        scratch_types=[
            pltpu.VMEM_SHARED((_NP,), jnp.float32),
            pltpu.VMEM_SHARED((_NP,), jnp.float32),
            pltpu.VMEM((_L,), jnp.int32),
            pltpu.VMEM((_L,), jnp.int32),
            pltpu.VMEM((_L,), jnp.float32),
        ],
    )(_sc_degrees_body)


def _sc_degrees_body(src_hbm, dst_hbm, ones_hbm, zeros_hbm, outd_hbm, ind_hbm,
                     oacc, iacc, idx_s, idx_d, ones_v):
    c = lax.axis_index("c")
    s = lax.axis_index("s")
    wid = s * _NC + c
    z0 = s * _NPT
    pltpu.sync_copy(zeros_hbm.at[pl.ds(z0, _NPT)], oacc.at[pl.ds(z0, _NPT)])
    pltpu.sync_copy(zeros_hbm.at[pl.ds(z0, _NPT)], iacc.at[pl.ds(z0, _NPT)])
    pltpu.sync_copy(ones_hbm, ones_v)
    plsc.subcore_barrier()
    base = wid * _RPW
    cnt = lax.max(0, lax.min(_RPW, _R - base))

    def body(r, carry):
        row = base + r
        pltpu.sync_copy(src_hbm.at[row], idx_s)
        pltpu.sync_copy(dst_hbm.at[row], idx_d)
        pltpu.sync_copy(ones_v, oacc.at[idx_s], add=True)
        pltpu.sync_copy(ones_v, iacc.at[idx_d], add=True)
        return carry

    lax.fori_loop(0, cnt, body, 0)
    plsc.subcore_barrier()
    pltpu.sync_copy(oacc.at[pl.ds(z0, _NPT)], outd_hbm.at[c, pl.ds(z0, _NPT)])
    pltpu.sync_copy(iacc.at[pl.ds(z0, _NPT)], ind_hbm.at[c, pl.ds(z0, _NPT)])


@functools.cache
def _make_sc_propagate(d):
    def _sc_propagate(src_hbm, dst_hbm, hw_hbm, zeros_hbm, out_hbm,
                      acc, idx_s, idx_d, rows_v, sem):
        c = lax.axis_index("c")
        s = lax.axis_index("s")
        wid = s * _NC + c
        z0 = s * _NPT
        pltpu.sync_copy(zeros_hbm.at[pl.ds(z0, _NPT)], acc.at[pl.ds(z0, _NPT)])
        plsc.subcore_barrier()
        base = wid * _RPW
        cnt = lax.max(0, lax.min(_RPW, _R - base))

        def body(r, carry):
            row = base + r
            pltpu.sync_copy(src_hbm.at[row], idx_s)
            pltpu.sync_copy(dst_hbm.at[row], idx_d)
            pltpu.async_copy(hw_hbm.at[idx_s], rows_v, sem).wait()
            pltpu.sync_copy(rows_v, acc.at[idx_d], add=True)
            return carry

        lax.fori_loop(0, cnt, body, 0)
        plsc.subcore_barrier()
        pltpu.sync_copy(acc.at[pl.ds(z0, _NPT)], out_hbm.at[c, pl.ds(z0, _NPT)])

    return functools.partial(
        pl.kernel,
        out_type=jax.ShapeDtypeStruct((_NC, _NP, d), jnp.float32),
        mesh=_sc_mesh(),
        scratch_types=[
            pltpu.VMEM_SHARED((_NP, d), jnp.float32),
            pltpu.VMEM((_L,), jnp.int32),
            pltpu.VMEM((_L,), jnp.int32),
            pltpu.VMEM((_L, d), jnp.float32),
            pltpu.SemaphoreType.DMA,
        ],
        compiler_params=pltpu.CompilerParams(use_tc_tiling_on_sc=False),
    )(_sc_propagate)


def _norms(degs):
    outd = degs[:, 0, 0:1] + degs[:, 1, 0:1]
    ind = degs[:, 0, 1:2] + degs[:, 1, 1:2]
    ns = lax.rsqrt(jnp.maximum(outd, 1.0))
    nd = lax.rsqrt(jnp.maximum(ind, 1.0))
    return ns, nd


def _tc_l1_body(feat_ref, degs_ref, w1_ref, o_ref):
    ns, _ = _norms(degs_ref[...])
    x = feat_ref[...] * ns
    o_ref[...] = jnp.dot(x, w1_ref[...], preferred_element_type=jnp.float32)


def _tc_l2_body(aggp_ref, degs_ref, w2_ref, b1_ref, o_ref):
    p = aggp_ref[...]
    ns, nd = _norms(degs_ref[...])
    h1 = jnp.maximum((p[0] + p[1]) * nd + b1_ref[...], 0.0)
    o_ref[...] = jnp.dot(h1 * ns, w2_ref[...],
                         preferred_element_type=jnp.float32)


def _tc_head_body(aggp_ref, degs_ref, b2_ref, wo_ref, bo_ref, o_ref, acc_ref):
    i = pl.program_id(0)

    @pl.when(i == 0)
    def _():
        acc_ref[...] = jnp.zeros_like(acc_ref)

    p = aggp_ref[...]
    _, nd = _norms(degs_ref[...])
    h2 = jnp.maximum((p[0] + p[1]) * nd + b2_ref[...], 0.0)
    rows = i * _BLK + lax.broadcasted_iota(jnp.int32, (_BLK, 1), 0)
    h2 = jnp.where(rows < _N, h2, 0.0)
    acc_ref[0:1, 0:4] += jnp.sum(h2, axis=0, keepdims=True)

    @pl.when(i == _GRID - 1)
    def _():
        g = acc_ref[0:1, 0:4] * (1.0 / _N)
        z = jnp.dot(g, wo_ref[...],
                    preferred_element_type=jnp.float32) + bo_ref[...]
        o_ref[...] = 1.0 / (1.0 + jnp.exp(-z))


def _tc_layer1(feat_p, degs_t, w1):
    return pl.pallas_call(
        _tc_l1_body,
        grid=(_GRID,),
        in_specs=[
            pl.BlockSpec((_BLK, 10), lambda i: (i, 0)),
            pl.BlockSpec((_BLK, 2, 2), lambda i: (i, 0, 0)),
            pl.BlockSpec((10, 8), lambda i: (0, 0)),
        ],
        out_specs=pl.BlockSpec((_BLK, 8), lambda i: (i, 0)),
        out_shape=jax.ShapeDtypeStruct((_NP, 8), jnp.float32),
    )(feat_p, degs_t, w1)


def _tc_layer2(agg1, degs_t, w2, b1):
    return pl.pallas_call(
        _tc_l2_body,
        grid=(_GRID,),
        in_specs=[
            pl.BlockSpec((_NC, _BLK, 8), lambda i: (0, i, 0)),
            pl.BlockSpec((_BLK, 2, 2), lambda i: (i, 0, 0)),
            pl.BlockSpec((8, 4), lambda i: (0, 0)),
            pl.BlockSpec((1, 8), lambda i: (0, 0)),
        ],
        out_specs=pl.BlockSpec((_BLK, 4), lambda i: (i, 0)),
        out_shape=jax.ShapeDtypeStruct((_NP, 4), jnp.float32),
    )(agg1, degs_t, w2, b1)


def _tc_head(agg2, degs_t, b2, wo, bo):
    return pl.pallas_call(
        _tc_head_body,
        grid=(_GRID,),
        in_specs=[
            pl.BlockSpec((_NC, _BLK, 4), lambda i: (0, i, 0)),
            pl.BlockSpec((_BLK, 2, 2), lambda i: (i, 0, 0)),
            pl.BlockSpec((1, 4), lambda i: (0, 0)),
            pl.BlockSpec((4, 1), lambda i: (0, 0)),
            pl.BlockSpec((1, 1), lambda i: (0, 0)),
        ],
        out_specs=pl.BlockSpec((1, 1), lambda i: (0, 0)),
        out_shape=jax.ShapeDtypeStruct((1, 1), jnp.float32),
        scratch_shapes=[pltpu.VMEM((8, 128), jnp.float32)],
    )(agg2, degs_t, b2, wo, bo)


def kernel(features, edge_index, W1, b1, W2, b2, Wo, bo):
    src2 = edge_index[0].reshape(_R, _L)
    dst2 = edge_index[1].reshape(_R, _L)
    feat_p = jnp.zeros((_NP, 10), jnp.float32).at[:_N].set(features)
    ones_l = jnp.ones((_L,), jnp.float32)
    z1 = jnp.zeros((_NP,), jnp.float32)
    z8 = jnp.zeros((_NP, 8), jnp.float32)
    z4 = jnp.zeros((_NP, 4), jnp.float32)

    outd, ind = _get_sc_degrees()(src2, dst2, ones_l, z1)
    degs_t = jnp.stack((outd.T, ind.T), axis=-1)

    hw1 = _tc_layer1(feat_p, degs_t, W1)
    agg1 = _make_sc_propagate(8)(src2, dst2, hw1, z8)
    hw2 = _tc_layer2(agg1, degs_t, W2, b1.reshape(1, 8))
    agg2 = _make_sc_propagate(4)(src2, dst2, hw2, z4)
    out = _tc_head(agg2, degs_t, b2.reshape(1, 4), Wo, bo.reshape(1, 1))
    return out.reshape(-1)


# speedup vs baseline: 39.0640x; 3.4338x over previous
import functools

import jax
import jax.numpy as jnp
from jax import lax
from jax.experimental import pallas as pl
from jax.experimental.pallas import tpu as pltpu
from jax.experimental.pallas import tpu_sc as plsc

_N = 100000
_E = 6400000
_L = 128
_R = _E // _L
_NC = 2
_NS = 16
_NW = _NC * _NS
_RPW = -(-_R // _NW)
_KB = 8
_NB = _R // _KB
_BPW = -(-_NB // _NW)
_BLK = 1024
_NP = ((_N + _BLK - 1) // _BLK) * _BLK
_GRID = _NP // _BLK
_NPT = _NP // _NS


def _sc_mesh():
    return plsc.VectorSubcoreMesh(core_axis_name="c", subcore_axis_name="s")


@functools.cache
def _get_sc_degrees():
    return functools.partial(
        pl.kernel,
        out_type=[jax.ShapeDtypeStruct((_NC, _NP), jnp.float32),
                  jax.ShapeDtypeStruct((_NC, _NP), jnp.float32)],
        mesh=_sc_mesh(),
        scratch_types=[
            pltpu.VMEM_SHARED((_NP,), jnp.float32),
            pltpu.VMEM_SHARED((_NP,), jnp.float32),
            pltpu.VMEM((_KB, _L), jnp.int32),
            pltpu.VMEM((_KB, _L), jnp.int32),
            pltpu.VMEM((_L,), jnp.float32),
            pltpu.SemaphoreType.DMA,
            pltpu.SemaphoreType.DMA,
        ],
    )(_sc_degrees_body)


def _sc_degrees_body(src_hbm, dst_hbm, ones_hbm, zeros_hbm, outd_hbm, ind_hbm,
                     oacc, iacc, idx_s, idx_d, ones_v, si, ss):
    c = lax.axis_index("c")
    s = lax.axis_index("s")
    wid = s * _NC + c
    z0 = s * _NPT
    pltpu.sync_copy(zeros_hbm.at[pl.ds(z0, _NPT)], oacc.at[pl.ds(z0, _NPT)])
    pltpu.sync_copy(zeros_hbm.at[pl.ds(z0, _NPT)], iacc.at[pl.ds(z0, _NPT)])
    pltpu.sync_copy(ones_hbm, ones_v)
    plsc.subcore_barrier()
    base = wid * _BPW
    nblk = lax.max(0, lax.min(_BPW, _NB - base))

    def body(t, carry):
        row = (base + t) * _KB
        di_s = pltpu.async_copy(src_hbm.at[pl.ds(row, _KB)], idx_s, si)
        di_d = pltpu.async_copy(dst_hbm.at[pl.ds(row, _KB)], idx_d, si)
        di_s.wait()
        di_d.wait()
        scs = []
        for j in range(_KB):
            scs.append(pltpu.async_copy(ones_v, oacc.at[idx_s.at[j]], ss,
                                        add=True))
            scs.append(pltpu.async_copy(ones_v, iacc.at[idx_d.at[j]], ss,
                                        add=True))
        for d_ in scs:
            d_.wait()
        return carry

    lax.fori_loop(0, nblk, body, 0)
    plsc.subcore_barrier()
    pltpu.sync_copy(oacc.at[pl.ds(z0, _NPT)], outd_hbm.at[c, pl.ds(z0, _NPT)])
    pltpu.sync_copy(iacc.at[pl.ds(z0, _NPT)], ind_hbm.at[c, pl.ds(z0, _NPT)])


@functools.cache
def _make_sc_propagate(d):
    def _sc_propagate(src_hbm, dst_hbm, hw_hbm, zeros_hbm, out_hbm,
                      acc, idx_s, idx_d, rows_v, si, sg, ss):
        c = lax.axis_index("c")
        s = lax.axis_index("s")
        wid = s * _NC + c
        z0 = s * _NPT
        pltpu.sync_copy(zeros_hbm.at[pl.ds(z0, _NPT)], acc.at[pl.ds(z0, _NPT)])
        plsc.subcore_barrier()
        base = wid * _BPW
        nblk = lax.max(0, lax.min(_BPW, _NB - base))

        def body(t, carry):
            row = (base + t) * _KB
            di_s = pltpu.async_copy(src_hbm.at[pl.ds(row, _KB)], idx_s, si)
            di_d = pltpu.async_copy(dst_hbm.at[pl.ds(row, _KB)], idx_d, si)
            di_s.wait()
            gs = []
            for j in range(_KB):
                gs.append(pltpu.async_copy(hw_hbm.at[idx_s.at[j]],
                                           rows_v.at[pl.ds(j * _L, _L)], sg))
            di_d.wait()
            scs = []
            for j in range(_KB):
                gs[j].wait()
                scs.append(pltpu.async_copy(rows_v.at[pl.ds(j * _L, _L)],
                                            acc.at[idx_d.at[j]], ss,
                                            add=True))
            for d_ in scs:
                d_.wait()
            return carry

        lax.fori_loop(0, nblk, body, 0)
        plsc.subcore_barrier()
        pltpu.sync_copy(acc.at[pl.ds(z0, _NPT)], out_hbm.at[c, pl.ds(z0, _NPT)])

    return functools.partial(
        pl.kernel,
        out_type=jax.ShapeDtypeStruct((_NC, _NP, d), jnp.float32),
        mesh=_sc_mesh(),
        scratch_types=[
            pltpu.VMEM_SHARED((_NP, d), jnp.float32),
            pltpu.VMEM((_KB, _L), jnp.int32),
            pltpu.VMEM((_KB, _L), jnp.int32),
            pltpu.VMEM((_KB * _L, d), jnp.float32),
            pltpu.SemaphoreType.DMA,
            pltpu.SemaphoreType.DMA,
            pltpu.SemaphoreType.DMA,
        ],
        compiler_params=pltpu.CompilerParams(use_tc_tiling_on_sc=False),
    )(_sc_propagate)


def _norms(degs):
    outd = degs[:, 0, 0:1] + degs[:, 1, 0:1]
    ind = degs[:, 0, 1:2] + degs[:, 1, 1:2]
    ns = lax.rsqrt(jnp.maximum(outd, 1.0))
    nd = lax.rsqrt(jnp.maximum(ind, 1.0))
    return ns, nd


def _tc_l1_body(feat_ref, degs_ref, w1_ref, o_ref):
    ns, _ = _norms(degs_ref[...])
    x = feat_ref[...] * ns
    o_ref[...] = jnp.dot(x, w1_ref[...], preferred_element_type=jnp.float32)


def _tc_l2_body(aggp_ref, degs_ref, w2_ref, b1_ref, o_ref):
    p = aggp_ref[...]
    ns, nd = _norms(degs_ref[...])
    h1 = jnp.maximum((p[0] + p[1]) * nd + b1_ref[...], 0.0)
    o_ref[...] = jnp.dot(h1 * ns, w2_ref[...],
                         preferred_element_type=jnp.float32)


def _tc_head_body(aggp_ref, degs_ref, b2_ref, wo_ref, bo_ref, o_ref, acc_ref):
    i = pl.program_id(0)

    @pl.when(i == 0)
    def _():
        acc_ref[...] = jnp.zeros_like(acc_ref)

    p = aggp_ref[...]
    _, nd = _norms(degs_ref[...])
    h2 = jnp.maximum((p[0] + p[1]) * nd + b2_ref[...], 0.0)
    rows = i * _BLK + lax.broadcasted_iota(jnp.int32, (_BLK, 1), 0)
    h2 = jnp.where(rows < _N, h2, 0.0)
    acc_ref[0:1, 0:4] += jnp.sum(h2, axis=0, keepdims=True)

    @pl.when(i == _GRID - 1)
    def _():
        g = acc_ref[0:1, 0:4] * (1.0 / _N)
        z = jnp.dot(g, wo_ref[...],
                    preferred_element_type=jnp.float32) + bo_ref[...]
        o_ref[...] = 1.0 / (1.0 + jnp.exp(-z))


def _tc_layer1(feat_p, degs_t, w1):
    return pl.pallas_call(
        _tc_l1_body,
        grid=(_GRID,),
        in_specs=[
            pl.BlockSpec((_BLK, 10), lambda i: (i, 0)),
            pl.BlockSpec((_BLK, 2, 2), lambda i: (i, 0, 0)),
            pl.BlockSpec((10, 8), lambda i: (0, 0)),
        ],
        out_specs=pl.BlockSpec((_BLK, 8), lambda i: (i, 0)),
        out_shape=jax.ShapeDtypeStruct((_NP, 8), jnp.float32),
    )(feat_p, degs_t, w1)


def _tc_layer2(agg1, degs_t, w2, b1):
    return pl.pallas_call(
        _tc_l2_body,
        grid=(_GRID,),
        in_specs=[
            pl.BlockSpec((_NC, _BLK, 8), lambda i: (0, i, 0)),
            pl.BlockSpec((_BLK, 2, 2), lambda i: (i, 0, 0)),
            pl.BlockSpec((8, 4), lambda i: (0, 0)),
            pl.BlockSpec((1, 8), lambda i: (0, 0)),
        ],
        out_specs=pl.BlockSpec((_BLK, 4), lambda i: (i, 0)),
        out_shape=jax.ShapeDtypeStruct((_NP, 4), jnp.float32),
    )(agg1, degs_t, w2, b1)


def _tc_head(agg2, degs_t, b2, wo, bo):
    return pl.pallas_call(
        _tc_head_body,
        grid=(_GRID,),
        in_specs=[
            pl.BlockSpec((_NC, _BLK, 4), lambda i: (0, i, 0)),
            pl.BlockSpec((_BLK, 2, 2), lambda i: (i, 0, 0)),
            pl.BlockSpec((1, 4), lambda i: (0, 0)),
            pl.BlockSpec((4, 1), lambda i: (0, 0)),
            pl.BlockSpec((1, 1), lambda i: (0, 0)),
        ],
        out_specs=pl.BlockSpec((1, 1), lambda i: (0, 0)),
        out_shape=jax.ShapeDtypeStruct((1, 1), jnp.float32),
        scratch_shapes=[pltpu.VMEM((8, 128), jnp.float32)],
    )(agg2, degs_t, b2, wo, bo)


def kernel(features, edge_index, W1, b1, W2, b2, Wo, bo):
    src2 = edge_index[0].reshape(_R, _L)
    dst2 = edge_index[1].reshape(_R, _L)
    feat_p = jnp.zeros((_NP, 10), jnp.float32).at[:_N].set(features)
    ones_l = jnp.ones((_L,), jnp.float32)
    z1 = jnp.zeros((_NP,), jnp.float32)
    z8 = jnp.zeros((_NP, 8), jnp.float32)
    z4 = jnp.zeros((_NP, 4), jnp.float32)

    outd, ind = _get_sc_degrees()(src2, dst2, ones_l, z1)
    degs_t = jnp.stack((outd.T, ind.T), axis=-1)

    hw1 = _tc_layer1(feat_p, degs_t, W1)
    agg1 = _make_sc_propagate(8)(src2, dst2, hw1, z8)
    hw2 = _tc_layer2(agg1, degs_t, W2, b1.reshape(1, 8))
    agg2 = _make_sc_propagate(4)(src2, dst2, hw2, z4)
    out = _tc_head(agg2, degs_t, b2.reshape(1, 4), Wo, bo.reshape(1, 1))
    return out.reshape(-1)


# speedup vs baseline: 42.9399x; 1.0992x over previous
import functools

import jax
import jax.numpy as jnp
from jax import lax
from jax.experimental import pallas as pl
from jax.experimental.pallas import tpu as pltpu
from jax.experimental.pallas import tpu_sc as plsc

_N = 100000
_E = 6400000
_L = 128
_R = _E // _L
_NC = 2
_NS = 16
_NW = _NC * _NS
_RPW = -(-_R // _NW)
_KB = 8
_NB = _R // _KB
_BPW = -(-_NB // _NW)
_BLK = 4096
_NP = 100096
_GRID = -(-_NP // _BLK)
_NPT = _NP // _NS


def _sc_mesh():
    return plsc.VectorSubcoreMesh(core_axis_name="c", subcore_axis_name="s")


@functools.cache
def _get_sc_degrees():
    return functools.partial(
        pl.kernel,
        out_type=[jax.ShapeDtypeStruct((_NC, _NP), jnp.float32),
                  jax.ShapeDtypeStruct((_NC, _NP), jnp.float32)],
        mesh=_sc_mesh(),
        scratch_types=[
            pltpu.VMEM_SHARED((_NP,), jnp.float32),
            pltpu.VMEM_SHARED((_NP,), jnp.float32),
            pltpu.VMEM((_KB, _L), jnp.int32),
            pltpu.VMEM((_KB, _L), jnp.int32),
            pltpu.VMEM((_L,), jnp.float32),
            pltpu.SemaphoreType.DMA,
            pltpu.SemaphoreType.DMA,
        ],
        compiler_params=pltpu.CompilerParams(use_tc_tiling_on_sc=False),
    )(_sc_degrees_body)


def _sc_degrees_body(src_hbm, dst_hbm, ones_hbm, zeros_hbm, outd_hbm, ind_hbm,
                     oacc, iacc, idx_s, idx_d, ones_v, si, ss):
    c = lax.axis_index("c")
    s = lax.axis_index("s")
    wid = s * _NC + c
    z0 = s * _NPT
    pltpu.sync_copy(zeros_hbm.at[pl.ds(z0, _NPT)], oacc.at[pl.ds(z0, _NPT)])
    pltpu.sync_copy(zeros_hbm.at[pl.ds(z0, _NPT)], iacc.at[pl.ds(z0, _NPT)])
    pltpu.sync_copy(ones_hbm, ones_v)
    plsc.subcore_barrier()
    base = wid * _BPW
    nblk = lax.max(0, lax.min(_BPW, _NB - base))

    def body(t, carry):
        row = (base + t) * _KB
        di_s = pltpu.async_copy(src_hbm.at[pl.ds(row, _KB)], idx_s, si)
        di_d = pltpu.async_copy(dst_hbm.at[pl.ds(row, _KB)], idx_d, si)
        di_s.wait()
        di_d.wait()
        scs = []
        for j in range(_KB):
            scs.append(pltpu.async_copy(ones_v, oacc.at[idx_s.at[j]], ss,
                                        add=True))
            scs.append(pltpu.async_copy(ones_v, iacc.at[idx_d.at[j]], ss,
                                        add=True))
        for d_ in scs:
            d_.wait()
        return carry

    lax.fori_loop(0, nblk, body, 0)
    plsc.subcore_barrier()
    pltpu.sync_copy(oacc.at[pl.ds(z0, _NPT)], outd_hbm.at[c, pl.ds(z0, _NPT)])
    pltpu.sync_copy(iacc.at[pl.ds(z0, _NPT)], ind_hbm.at[c, pl.ds(z0, _NPT)])


@functools.cache
def _make_sc_propagate(d):
    def _sc_propagate(src_hbm, dst_hbm, hw_hbm, zeros_hbm, out_hbm,
                      acc, idx_s, idx_d, rows_v, si, sg, ss):
        c = lax.axis_index("c")
        s = lax.axis_index("s")
        wid = s * _NC + c
        z0 = s * _NPT
        pltpu.sync_copy(zeros_hbm.at[pl.ds(z0, _NPT)], acc.at[pl.ds(z0, _NPT)])
        plsc.subcore_barrier()
        base = wid * _BPW
        nblk = lax.max(0, lax.min(_BPW, _NB - base))

        def body(t, carry):
            row = (base + t) * _KB
            di_s = pltpu.async_copy(src_hbm.at[pl.ds(row, _KB)], idx_s, si)
            di_d = pltpu.async_copy(dst_hbm.at[pl.ds(row, _KB)], idx_d, si)
            di_s.wait()
            gs = []
            for j in range(_KB):
                gs.append(pltpu.async_copy(hw_hbm.at[idx_s.at[j]],
                                           rows_v.at[pl.ds(j * _L, _L)], sg))
            di_d.wait()
            scs = []
            for j in range(_KB):
                gs[j].wait()
                scs.append(pltpu.async_copy(rows_v.at[pl.ds(j * _L, _L)],
                                            acc.at[idx_d.at[j]], ss,
                                            add=True))
            for d_ in scs:
                d_.wait()
            return carry

        lax.fori_loop(0, nblk, body, 0)
        plsc.subcore_barrier()
        pltpu.sync_copy(acc.at[pl.ds(z0, _NPT)], out_hbm.at[c, pl.ds(z0, _NPT)])

    return functools.partial(
        pl.kernel,
        out_type=jax.ShapeDtypeStruct((_NC, _NP, d), jnp.float32),
        mesh=_sc_mesh(),
        scratch_types=[
            pltpu.VMEM_SHARED((_NP, d), jnp.float32),
            pltpu.VMEM((_KB, _L), jnp.int32),
            pltpu.VMEM((_KB, _L), jnp.int32),
            pltpu.VMEM((_KB * _L, d), jnp.float32),
            pltpu.SemaphoreType.DMA,
            pltpu.SemaphoreType.DMA,
            pltpu.SemaphoreType.DMA,
        ],
        compiler_params=pltpu.CompilerParams(use_tc_tiling_on_sc=False),
    )(_sc_propagate)


def _tc_l1_body(feat_ref, degs_ref, w1_ref, o_ref, ns_ref, nd_ref):
    degs = degs_ref[...]
    outd = degs[:, 0, 0:1] + degs[:, 1, 0:1]
    ind = degs[:, 0, 1:2] + degs[:, 1, 1:2]
    ns = lax.rsqrt(jnp.maximum(outd, 1.0))
    nd = lax.rsqrt(jnp.maximum(ind, 1.0))
    ns_ref[...] = ns
    nd_ref[...] = nd
    x = feat_ref[...] * ns
    o_ref[...] = jnp.dot(x, w1_ref[...], preferred_element_type=jnp.float32)


def _tc_l2_body(aggp_ref, ns_ref, nd_ref, w2_ref, b1_ref, o_ref):
    p = aggp_ref[...]
    h1 = jnp.maximum((p[0] + p[1]) * nd_ref[...] + b1_ref[...], 0.0)
    o_ref[...] = jnp.dot(h1 * ns_ref[...], w2_ref[...],
                         preferred_element_type=jnp.float32)


def _tc_head_body(aggp_ref, nd_ref, b2_ref, wo_ref, bo_ref, o_ref, acc_ref):
    i = pl.program_id(0)

    @pl.when(i == 0)
    def _():
        acc_ref[...] = jnp.zeros_like(acc_ref)

    p = aggp_ref[...]
    h2 = jnp.maximum((p[0] + p[1]) * nd_ref[...] + b2_ref[...], 0.0)
    rows = i * _BLK + lax.broadcasted_iota(jnp.int32, (_BLK, 1), 0)
    h2 = jnp.where(rows < _N, h2, 0.0)
    acc_ref[0:1, 0:4] += jnp.sum(h2, axis=0, keepdims=True)

    @pl.when(i == _GRID - 1)
    def _():
        g = acc_ref[0:1, 0:4] * (1.0 / _N)
        z = jnp.dot(g, wo_ref[...],
                    preferred_element_type=jnp.float32) + bo_ref[...]
        o_ref[...] = 1.0 / (1.0 + jnp.exp(-z))


def _tc_layer1(feat, degs_t, w1):
    return pl.pallas_call(
        _tc_l1_body,
        grid=(_GRID,),
        in_specs=[
            pl.BlockSpec((_BLK, 10), lambda i: (i, 0)),
            pl.BlockSpec((_BLK, 2, 2), lambda i: (i, 0, 0)),
            pl.BlockSpec((10, 8), lambda i: (0, 0)),
        ],
        out_specs=[
            pl.BlockSpec((_BLK, 8), lambda i: (i, 0)),
            pl.BlockSpec((_BLK, 1), lambda i: (i, 0)),
            pl.BlockSpec((_BLK, 1), lambda i: (i, 0)),
        ],
        out_shape=[
            jax.ShapeDtypeStruct((_NP, 8), jnp.float32),
            jax.ShapeDtypeStruct((_NP, 1), jnp.float32),
            jax.ShapeDtypeStruct((_NP, 1), jnp.float32),
        ],
    )(feat, degs_t, w1)


def _tc_layer2(agg1, ns, nd, w2, b1):
    return pl.pallas_call(
        _tc_l2_body,
        grid=(_GRID,),
        in_specs=[
            pl.BlockSpec((_NC, _BLK, 8), lambda i: (0, i, 0)),
            pl.BlockSpec((_BLK, 1), lambda i: (i, 0)),
            pl.BlockSpec((_BLK, 1), lambda i: (i, 0)),
            pl.BlockSpec((8, 4), lambda i: (0, 0)),
            pl.BlockSpec((1, 8), lambda i: (0, 0)),
        ],
        out_specs=pl.BlockSpec((_BLK, 4), lambda i: (i, 0)),
        out_shape=jax.ShapeDtypeStruct((_NP, 4), jnp.float32),
    )(agg1, ns, nd, w2, b1)


def _tc_head(agg2, nd, b2, wo, bo):
    return pl.pallas_call(
        _tc_head_body,
        grid=(_GRID,),
        in_specs=[
            pl.BlockSpec((_NC, _BLK, 4), lambda i: (0, i, 0)),
            pl.BlockSpec((_BLK, 1), lambda i: (i, 0)),
            pl.BlockSpec((1, 4), lambda i: (0, 0)),
            pl.BlockSpec((4, 1), lambda i: (0, 0)),
            pl.BlockSpec((1, 1), lambda i: (0, 0)),
        ],
        out_specs=pl.BlockSpec((1, 1), lambda i: (0, 0)),
        out_shape=jax.ShapeDtypeStruct((1, 1), jnp.float32),
        scratch_shapes=[pltpu.VMEM((8, 128), jnp.float32)],
    )(agg2, nd, b2, wo, bo)


def kernel(features, edge_index, W1, b1, W2, b2, Wo, bo):
    src2 = edge_index[0].reshape(_R, _L)
    dst2 = edge_index[1].reshape(_R, _L)
    ones_l = jnp.ones((_L,), jnp.float32)
    z1 = jnp.zeros((_NP,), jnp.float32)
    z8 = jnp.zeros((_NP, 8), jnp.float32)
    z4 = jnp.zeros((_NP, 4), jnp.float32)

    outd, ind = _get_sc_degrees()(src2, dst2, ones_l, z1)
    degs_t = jnp.stack((outd.T, ind.T), axis=-1)

    hw1, ns, nd = _tc_layer1(features, degs_t, W1)
    agg1 = _make_sc_propagate(8)(src2, dst2, hw1, z8)
    hw2 = _tc_layer2(agg1, ns, nd, W2, b1.reshape(1, 8))
    agg2 = _make_sc_propagate(4)(src2, dst2, hw2, z4)
    out = _tc_head(agg2, nd, b2.reshape(1, 4), Wo, bo.reshape(1, 1))
    return out.reshape(-1)


# speedup vs baseline: 45.2455x; 1.0537x over previous
import functools

import jax
import jax.numpy as jnp
from jax import lax
from jax.experimental import pallas as pl
from jax.experimental.pallas import tpu as pltpu
from jax.experimental.pallas import tpu_sc as plsc

_N = 100000
_E = 6400000
_L = 128
_R = _E // _L
_NC = 2
_NS = 16
_NW = _NC * _NS
_RPW = -(-_R // _NW)
_KB = 10
_NB = _R // _KB
_BPW = -(-_NB // _NW)
_BLK = 4096
_NP = 100096
_GRID = -(-_NP // _BLK)
_NPT = _NP // _NS


def _sc_mesh():
    return plsc.VectorSubcoreMesh(core_axis_name="c", subcore_axis_name="s")


@functools.cache
def _get_sc_degrees():
    return functools.partial(
        pl.kernel,
        out_type=[jax.ShapeDtypeStruct((_NC, _NP), jnp.float32),
                  jax.ShapeDtypeStruct((_NC, _NP), jnp.float32)],
        mesh=_sc_mesh(),
        scratch_types=[
            pltpu.VMEM_SHARED((_NP,), jnp.float32),
            pltpu.VMEM_SHARED((_NP,), jnp.float32),
            pltpu.VMEM((_KB, _L), jnp.int32),
            pltpu.VMEM((_KB, _L), jnp.int32),
            pltpu.VMEM((_L,), jnp.float32),
            pltpu.SemaphoreType.DMA,
            pltpu.SemaphoreType.DMA,
        ],
        compiler_params=pltpu.CompilerParams(use_tc_tiling_on_sc=False),
    )(_sc_degrees_body)


def _sc_degrees_body(src_hbm, dst_hbm, ones_hbm, zeros_hbm, outd_hbm, ind_hbm,
                     oacc, iacc, idx_s, idx_d, ones_v, si, ss):
    c = lax.axis_index("c")
    s = lax.axis_index("s")
    wid = s * _NC + c
    z0 = s * _NPT
    pltpu.sync_copy(zeros_hbm.at[pl.ds(z0, _NPT)], oacc.at[pl.ds(z0, _NPT)])
    pltpu.sync_copy(zeros_hbm.at[pl.ds(z0, _NPT)], iacc.at[pl.ds(z0, _NPT)])
    pltpu.sync_copy(ones_hbm, ones_v)
    plsc.subcore_barrier()
    base = wid * _BPW
    nblk = lax.max(0, lax.min(_BPW, _NB - base))

    def body(t, carry):
        row = (base + t) * _KB
        di_s = pltpu.async_copy(src_hbm.at[pl.ds(row, _KB)], idx_s, si)
        di_d = pltpu.async_copy(dst_hbm.at[pl.ds(row, _KB)], idx_d, si)
        di_s.wait()
        di_d.wait()
        scs = []
        for j in range(_KB):
            scs.append(pltpu.async_copy(ones_v, oacc.at[idx_s.at[j]], ss,
                                        add=True))
            scs.append(pltpu.async_copy(ones_v, iacc.at[idx_d.at[j]], ss,
                                        add=True))
        for d_ in scs:
            d_.wait()
        return carry

    lax.fori_loop(0, nblk, body, 0)
    plsc.subcore_barrier()
    pltpu.sync_copy(oacc.at[pl.ds(z0, _NPT)], outd_hbm.at[c, pl.ds(z0, _NPT)])
    pltpu.sync_copy(iacc.at[pl.ds(z0, _NPT)], ind_hbm.at[c, pl.ds(z0, _NPT)])


@functools.cache
def _make_sc_propagate(d):
    def _sc_propagate(src_hbm, dst_hbm, hw_hbm, zeros_hbm, out_hbm,
                      acc, idx_s, idx_d, rows_v, si, sg, ss):
        c = lax.axis_index("c")
        s = lax.axis_index("s")
        wid = s * _NC + c
        z0 = s * _NPT
        pltpu.sync_copy(zeros_hbm.at[pl.ds(z0, _NPT)], acc.at[pl.ds(z0, _NPT)])
        plsc.subcore_barrier()
        base = wid * _BPW
        nblk = lax.max(0, lax.min(_BPW, _NB - base))

        def body(t, carry):
            row = (base + t) * _KB
            di_s = pltpu.async_copy(src_hbm.at[pl.ds(row, _KB)], idx_s, si)
            di_d = pltpu.async_copy(dst_hbm.at[pl.ds(row, _KB)], idx_d, si)
            di_s.wait()
            gs = []
            for j in range(_KB):
                gs.append(pltpu.async_copy(hw_hbm.at[idx_s.at[j]],
                                           rows_v.at[pl.ds(j * _L, _L)], sg))
            di_d.wait()
            scs = []
            for j in range(_KB):
                gs[j].wait()
                scs.append(pltpu.async_copy(rows_v.at[pl.ds(j * _L, _L)],
                                            acc.at[idx_d.at[j]], ss,
                                            add=True))
            for d_ in scs:
                d_.wait()
            return carry

        lax.fori_loop(0, nblk, body, 0)
        plsc.subcore_barrier()
        pltpu.sync_copy(acc.at[pl.ds(z0, _NPT)], out_hbm.at[c, pl.ds(z0, _NPT)])

    return functools.partial(
        pl.kernel,
        out_type=jax.ShapeDtypeStruct((_NC, _NP, d), jnp.float32),
        mesh=_sc_mesh(),
        scratch_types=[
            pltpu.VMEM_SHARED((_NP, d), jnp.float32),
            pltpu.VMEM((_KB, _L), jnp.int32),
            pltpu.VMEM((_KB, _L), jnp.int32),
            pltpu.VMEM((_KB * _L, d), jnp.float32),
            pltpu.SemaphoreType.DMA,
            pltpu.SemaphoreType.DMA,
            pltpu.SemaphoreType.DMA,
        ],
        compiler_params=pltpu.CompilerParams(use_tc_tiling_on_sc=False),
    )(_sc_propagate)


def _tc_l1_body(feat_ref, degs_ref, w1_ref, o_ref, ns_ref, nd_ref):
    degs = degs_ref[...]
    outd = degs[:, 0, 0:1] + degs[:, 1, 0:1]
    ind = degs[:, 0, 1:2] + degs[:, 1, 1:2]
    ns = lax.rsqrt(jnp.maximum(outd, 1.0))
    nd = lax.rsqrt(jnp.maximum(ind, 1.0))
    ns_ref[...] = ns
    nd_ref[...] = nd
    x = feat_ref[...] * ns
    o_ref[...] = jnp.dot(x, w1_ref[...], preferred_element_type=jnp.float32)


def _tc_l2_body(aggp_ref, ns_ref, nd_ref, w2_ref, b1_ref, o_ref):
    p = aggp_ref[...]
    h1 = jnp.maximum((p[0] + p[1]) * nd_ref[...] + b1_ref[...], 0.0)
    o_ref[...] = jnp.dot(h1 * ns_ref[...], w2_ref[...],
                         preferred_element_type=jnp.float32)


def _tc_head_body(aggp_ref, nd_ref, b2_ref, wo_ref, bo_ref, o_ref, acc_ref):
    i = pl.program_id(0)

    @pl.when(i == 0)
    def _():
        acc_ref[...] = jnp.zeros_like(acc_ref)

    p = aggp_ref[...]
    h2 = jnp.maximum((p[0] + p[1]) * nd_ref[...] + b2_ref[...], 0.0)
    rows = i * _BLK + lax.broadcasted_iota(jnp.int32, (_BLK, 1), 0)
    h2 = jnp.where(rows < _N, h2, 0.0)
    acc_ref[0:1, 0:4] += jnp.sum(h2, axis=0, keepdims=True)

    @pl.when(i == _GRID - 1)
    def _():
        g = acc_ref[0:1, 0:4] * (1.0 / _N)
        z = jnp.dot(g, wo_ref[...],
                    preferred_element_type=jnp.float32) + bo_ref[...]
        o_ref[...] = 1.0 / (1.0 + jnp.exp(-z))


def _tc_layer1(feat, degs_t, w1):
    return pl.pallas_call(
        _tc_l1_body,
        grid=(_GRID,),
        in_specs=[
            pl.BlockSpec((_BLK, 10), lambda i: (i, 0)),
            pl.BlockSpec((_BLK, 2, 2), lambda i: (i, 0, 0)),
            pl.BlockSpec((10, 8), lambda i: (0, 0)),
        ],
        out_specs=[
            pl.BlockSpec((_BLK, 8), lambda i: (i, 0)),
            pl.BlockSpec((_BLK, 1), lambda i: (i, 0)),
            pl.BlockSpec((_BLK, 1), lambda i: (i, 0)),
        ],
        out_shape=[
            jax.ShapeDtypeStruct((_NP, 8), jnp.float32),
            jax.ShapeDtypeStruct((_NP, 1), jnp.float32),
            jax.ShapeDtypeStruct((_NP, 1), jnp.float32),
        ],
    )(feat, degs_t, w1)


def _tc_layer2(agg1, ns, nd, w2, b1):
    return pl.pallas_call(
        _tc_l2_body,
        grid=(_GRID,),
        in_specs=[
            pl.BlockSpec((_NC, _BLK, 8), lambda i: (0, i, 0)),
            pl.BlockSpec((_BLK, 1), lambda i: (i, 0)),
            pl.BlockSpec((_BLK, 1), lambda i: (i, 0)),
            pl.BlockSpec((8, 4), lambda i: (0, 0)),
            pl.BlockSpec((1, 8), lambda i: (0, 0)),
        ],
        out_specs=pl.BlockSpec((_BLK, 4), lambda i: (i, 0)),
        out_shape=jax.ShapeDtypeStruct((_NP, 4), jnp.float32),
    )(agg1, ns, nd, w2, b1)


def _tc_head(agg2, nd, b2, wo, bo):
    return pl.pallas_call(
        _tc_head_body,
        grid=(_GRID,),
        in_specs=[
            pl.BlockSpec((_NC, _BLK, 4), lambda i: (0, i, 0)),
            pl.BlockSpec((_BLK, 1), lambda i: (i, 0)),
            pl.BlockSpec((1, 4), lambda i: (0, 0)),
            pl.BlockSpec((4, 1), lambda i: (0, 0)),
            pl.BlockSpec((1, 1), lambda i: (0, 0)),
        ],
        out_specs=pl.BlockSpec((1, 1), lambda i: (0, 0)),
        out_shape=jax.ShapeDtypeStruct((1, 1), jnp.float32),
        scratch_shapes=[pltpu.VMEM((8, 128), jnp.float32)],
    )(agg2, nd, b2, wo, bo)


def kernel(features, edge_index, W1, b1, W2, b2, Wo, bo):
    src2 = edge_index[0].reshape(_R, _L)
    dst2 = edge_index[1].reshape(_R, _L)
    ones_l = jnp.ones((_L,), jnp.float32)
    z1 = jnp.zeros((_NP,), jnp.float32)
    z8 = jnp.zeros((_NP, 8), jnp.float32)
    z4 = jnp.zeros((_NP, 4), jnp.float32)

    outd, ind = _get_sc_degrees()(src2, dst2, ones_l, z1)
    degs_t = jnp.stack((outd.T, ind.T), axis=-1)

    hw1, ns, nd = _tc_layer1(features, degs_t, W1)
    agg1 = _make_sc_propagate(8)(src2, dst2, hw1, z8)
    hw2 = _tc_layer2(agg1, ns, nd, W2, b1.reshape(1, 8))
    agg2 = _make_sc_propagate(4)(src2, dst2, hw2, z4)
    out = _tc_head(agg2, nd, b2.reshape(1, 4), Wo, bo.reshape(1, 1))
    return out.reshape(-1)


# speedup vs baseline: 60.3100x; 1.3329x over previous
import functools

import jax
import jax.numpy as jnp
from jax import lax
from jax.experimental import pallas as pl
from jax.experimental.pallas import tpu as pltpu
from jax.experimental.pallas import tpu_sc as plsc

_N = 100000
_E = 6400000
_L = 128
_R = _E // _L
_NC = 2
_NS = 16
_NW = _NC * _NS
_RPW = -(-_R // _NW)
_KB = 16
_NB = _R // _KB
_BPW = -(-_NB // _NW)
_BLK = 4096
_NP = 100096
_GRID = -(-_NP // _BLK)
_NPT = _NP // _NS


def _sc_mesh():
    return plsc.VectorSubcoreMesh(core_axis_name="c", subcore_axis_name="s")


@functools.cache
def _get_sc_degrees():
    return functools.partial(
        pl.kernel,
        out_type=[jax.ShapeDtypeStruct((_NC, _NP), jnp.float32),
                  jax.ShapeDtypeStruct((_NC, _NP), jnp.float32)],
        mesh=_sc_mesh(),
        scratch_types=[
            pltpu.VMEM_SHARED((_NP,), jnp.float32),
            pltpu.VMEM_SHARED((_NP,), jnp.float32),
            pltpu.VMEM((_KB, _L), jnp.int32),
            pltpu.VMEM((_KB, _L), jnp.int32),
            pltpu.VMEM((_L,), jnp.float32),
            pltpu.SemaphoreType.DMA,
            pltpu.SemaphoreType.DMA,
        ],
        compiler_params=pltpu.CompilerParams(use_tc_tiling_on_sc=False),
    )(_sc_degrees_body)


def _sc_degrees_body(src_hbm, dst_hbm, ones_hbm, zeros_hbm, outd_hbm, ind_hbm,
                     oacc, iacc, idx_s, idx_d, ones_v, si, ss):
    c = lax.axis_index("c")
    s = lax.axis_index("s")
    wid = s * _NC + c
    z0 = s * _NPT
    pltpu.sync_copy(zeros_hbm.at[pl.ds(z0, _NPT)], oacc.at[pl.ds(z0, _NPT)])
    pltpu.sync_copy(zeros_hbm.at[pl.ds(z0, _NPT)], iacc.at[pl.ds(z0, _NPT)])
    pltpu.sync_copy(ones_hbm, ones_v)
    plsc.subcore_barrier()
    base = wid * _BPW
    nblk = lax.max(0, lax.min(_BPW, _NB - base))

    def body(t, carry):
        row = (base + t) * _KB
        di_s = pltpu.async_copy(src_hbm.at[pl.ds(row, _KB)], idx_s, si)
        di_d = pltpu.async_copy(dst_hbm.at[pl.ds(row, _KB)], idx_d, si)
        di_s.wait()
        di_d.wait()
        scs = []
        for j in range(_KB):
            scs.append(pltpu.async_copy(ones_v, oacc.at[idx_s.at[j]], ss,
                                        add=True))
            scs.append(pltpu.async_copy(ones_v, iacc.at[idx_d.at[j]], ss,
                                        add=True))
        for d_ in scs:
            d_.wait()
        return carry

    lax.fori_loop(0, nblk, body, 0)
    plsc.subcore_barrier()
    pltpu.sync_copy(oacc.at[pl.ds(z0, _NPT)], outd_hbm.at[c, pl.ds(z0, _NPT)])
    pltpu.sync_copy(iacc.at[pl.ds(z0, _NPT)], ind_hbm.at[c, pl.ds(z0, _NPT)])


@functools.cache
def _make_sc_propagate(d):
    def _sc_propagate(src_hbm, dst_hbm, hw_hbm, zeros_hbm, out_hbm,
                      acc, idx_s, idx_d, rows_v, si, sg, ss):
        c = lax.axis_index("c")
        s = lax.axis_index("s")
        wid = s * _NC + c
        z0 = s * _NPT
        pltpu.sync_copy(zeros_hbm.at[pl.ds(z0, _NPT)], acc.at[pl.ds(z0, _NPT)])
        plsc.subcore_barrier()
        base = wid * _BPW
        nblk = lax.max(0, lax.min(_BPW, _NB - base))

        def body(t, carry):
            row = (base + t) * _KB
            di_s = pltpu.async_copy(src_hbm.at[pl.ds(row, _KB)], idx_s, si)
            di_d = pltpu.async_copy(dst_hbm.at[pl.ds(row, _KB)], idx_d, si)
            di_s.wait()
            gs = []
            for j in range(_KB):
                gs.append(pltpu.async_copy(hw_hbm.at[idx_s.at[j]],
                                           rows_v.at[pl.ds(j * _L, _L)], sg))
            di_d.wait()
            scs = []
            for j in range(_KB):
                gs[j].wait()
                scs.append(pltpu.async_copy(rows_v.at[pl.ds(j * _L, _L)],
                                            acc.at[idx_d.at[j]], ss,
                                            add=True))
            for d_ in scs:
                d_.wait()
            return carry

        lax.fori_loop(0, nblk, body, 0)
        plsc.subcore_barrier()
        pltpu.sync_copy(acc.at[pl.ds(z0, _NPT)], out_hbm.at[c, pl.ds(z0, _NPT)])

    return functools.partial(
        pl.kernel,
        out_type=jax.ShapeDtypeStruct((_NC, _NP, d), jnp.float32),
        mesh=_sc_mesh(),
        scratch_types=[
            pltpu.VMEM_SHARED((_NP, d), jnp.float32),
            pltpu.VMEM((_KB, _L), jnp.int32),
            pltpu.VMEM((_KB, _L), jnp.int32),
            pltpu.VMEM((_KB * _L, d), jnp.float32),
            pltpu.SemaphoreType.DMA,
            pltpu.SemaphoreType.DMA,
            pltpu.SemaphoreType.DMA,
        ],
        compiler_params=pltpu.CompilerParams(use_tc_tiling_on_sc=False),
    )(_sc_propagate)


def _tc_l1_body(feat_ref, od_ref, id_ref, w1_ref, o_ref, ns_ref, nd_ref):
    ns = lax.rsqrt(jnp.maximum(od_ref[...], 1.0))
    nd = lax.rsqrt(jnp.maximum(id_ref[...], 1.0))
    ns_ref[...] = ns
    nd_ref[...] = nd
    x = feat_ref[...] * ns
    o_ref[...] = jnp.dot(x, w1_ref[...], preferred_element_type=jnp.float32)


def _tc_l2_body(aggp_ref, ns_ref, nd_ref, w2_ref, b1_ref, o_ref):
    p = aggp_ref[...]
    h1 = jnp.maximum((p[0] + p[1]) * nd_ref[...] + b1_ref[...], 0.0)
    o_ref[...] = jnp.dot(h1 * ns_ref[...], w2_ref[...],
                         preferred_element_type=jnp.float32)


def _tc_head_body(aggp_ref, nd_ref, b2_ref, wo_ref, bo_ref, o_ref, acc_ref):
    i = pl.program_id(0)

    @pl.when(i == 0)
    def _():
        acc_ref[...] = jnp.zeros_like(acc_ref)

    p = aggp_ref[...]
    h2 = jnp.maximum((p[0] + p[1]) * nd_ref[...] + b2_ref[...], 0.0)
    rows = i * _BLK + lax.broadcasted_iota(jnp.int32, (_BLK, 1), 0)
    h2 = jnp.where(rows < _N, h2, 0.0)
    acc_ref[0:1, 0:4] += jnp.sum(h2, axis=0, keepdims=True)

    @pl.when(i == _GRID - 1)
    def _():
        g = acc_ref[0:1, 0:4] * (1.0 / _N)
        z = jnp.dot(g, wo_ref[...],
                    preferred_element_type=jnp.float32) + bo_ref[...]
        o_ref[...] = 1.0 / (1.0 + jnp.exp(-z))


def _tc_layer1(feat, od, idg, w1):
    return pl.pallas_call(
        _tc_l1_body,
        grid=(_GRID,),
        in_specs=[
            pl.BlockSpec((_BLK, 10), lambda i: (i, 0)),
            pl.BlockSpec((_BLK, 1), lambda i: (i, 0)),
            pl.BlockSpec((_BLK, 1), lambda i: (i, 0)),
            pl.BlockSpec((10, 8), lambda i: (0, 0)),
        ],
        out_specs=[
            pl.BlockSpec((_BLK, 8), lambda i: (i, 0)),
            pl.BlockSpec((_BLK, 1), lambda i: (i, 0)),
            pl.BlockSpec((_BLK, 1), lambda i: (i, 0)),
        ],
        out_shape=[
            jax.ShapeDtypeStruct((_NP, 8), jnp.float32),
            jax.ShapeDtypeStruct((_NP, 1), jnp.float32),
            jax.ShapeDtypeStruct((_NP, 1), jnp.float32),
        ],
    )(feat, od, idg, w1)


def _tc_layer2(agg1, ns, nd, w2, b1):
    return pl.pallas_call(
        _tc_l2_body,
        grid=(_GRID,),
        in_specs=[
            pl.BlockSpec((_NC, _BLK, 8), lambda i: (0, i, 0)),
            pl.BlockSpec((_BLK, 1), lambda i: (i, 0)),
            pl.BlockSpec((_BLK, 1), lambda i: (i, 0)),
            pl.BlockSpec((8, 4), lambda i: (0, 0)),
            pl.BlockSpec((1, 8), lambda i: (0, 0)),
        ],
        out_specs=pl.BlockSpec((_BLK, 4), lambda i: (i, 0)),
        out_shape=jax.ShapeDtypeStruct((_NP, 4), jnp.float32),
    )(agg1, ns, nd, w2, b1)


def _tc_head(agg2, nd, b2, wo, bo):
    return pl.pallas_call(
        _tc_head_body,
        grid=(_GRID,),
        in_specs=[
            pl.BlockSpec((_NC, _BLK, 4), lambda i: (0, i, 0)),
            pl.BlockSpec((_BLK, 1), lambda i: (i, 0)),
            pl.BlockSpec((1, 4), lambda i: (0, 0)),
            pl.BlockSpec((4, 1), lambda i: (0, 0)),
            pl.BlockSpec((1, 1), lambda i: (0, 0)),
        ],
        out_specs=pl.BlockSpec((1, 1), lambda i: (0, 0)),
        out_shape=jax.ShapeDtypeStruct((1, 1), jnp.float32),
        scratch_shapes=[pltpu.VMEM((8, 128), jnp.float32)],
    )(agg2, nd, b2, wo, bo)


def kernel(features, edge_index, W1, b1, W2, b2, Wo, bo):
    src2 = edge_index[0].reshape(_R, _L)
    dst2 = edge_index[1].reshape(_R, _L)
    ones_l = jnp.ones((_L,), jnp.float32)
    z1 = jnp.zeros((_NP,), jnp.float32)
    z8 = jnp.zeros((_NP, 8), jnp.float32)
    z4 = jnp.zeros((_NP, 4), jnp.float32)

    outd, ind = _get_sc_degrees()(src2, dst2, ones_l, z1)
    od = (outd[0] + outd[1]).reshape(_NP, 1)
    idg = (ind[0] + ind[1]).reshape(_NP, 1)

    hw1, ns, nd = _tc_layer1(features, od, idg, W1)
    agg1 = _make_sc_propagate(8)(src2, dst2, hw1, z8)
    hw2 = _tc_layer2(agg1, ns, nd, W2, b1.reshape(1, 8))
    agg2 = _make_sc_propagate(4)(src2, dst2, hw2, z4)
    out = _tc_head(agg2, nd, b2.reshape(1, 4), Wo, bo.reshape(1, 1))
    return out.reshape(-1)
